# baseline (speedup 1.0000x reference)
import jax
import jax.numpy as jnp
from jax import lax
from jax.experimental import pallas as pl
from jax.experimental.pallas import tpu as pltpu

B = 1024
D = 32
BI = 128


def _body(idx_ref, *refs):
    col_refs = refs[:BI]
    eb_ref, out_ref = refs[BI], refs[BI + 1]
    i = pl.program_id(0)

    cols = []
    for t in range(BI):
        lane = lax.rem(idx_ref[i * BI + t], 128)
        onehot = lax.broadcasted_iota(jnp.int32, (D, 128), 1) == lane
        x = col_refs[t][...]
        cols.append(jnp.sum(jnp.where(onehot, x, 0.0), axis=1, keepdims=True))
    gT = jnp.concatenate(cols, axis=1)
    eye = (lax.broadcasted_iota(jnp.int32, (D, D), 0)
           == lax.broadcasted_iota(jnp.int32, (D, D), 1)).astype(jnp.float32)
    g = lax.dot_general(gT, eye, (((0,), (0,)), ((), ())),
                        preferred_element_type=jnp.float32)
    n = g * lax.rsqrt(jnp.sum(g * g, axis=1, keepdims=True))
    dots = jnp.dot(n, eb_ref[:, 0, :], preferred_element_type=jnp.float32)
    for d in range(D):
        ebd = eb_ref[d]
        ncol = n[:, d : d + 1]
        for s in range(BI // 8):
            out_ref[s * 8 : (s + 1) * 8, d, :] = (
                ebd - ncol[s * 8 : (s + 1) * 8, :] * dots[s * 8 : (s + 1) * 8, :]
            )


def kernel(batch_e_emb, batch_r_id, w_r):
    idx = batch_r_id.reshape(B).astype(jnp.int32)
    eT = batch_e_emb.T
    e_bcast = jnp.broadcast_to(eT[:, None, :], (D, 8, B))
    wT = w_r.T

    def col_spec(t):
        return pl.BlockSpec(
            (D, 128), lambda i, idx_ref, t=t: (0, idx_ref[i * BI + t] // 128)
        )

    grid_spec = pltpu.PrefetchScalarGridSpec(
        num_scalar_prefetch=1,
        grid=(B // BI,),
        in_specs=[col_spec(t) for t in range(BI)]
        + [pl.BlockSpec((D, 8, B), lambda i, idx_ref: (0, 0, 0))],
        out_specs=pl.BlockSpec((BI, D, B), lambda i, idx_ref: (i, 0, 0)),
        scratch_shapes=[],
    )
    out_idj = pl.pallas_call(
        _body,
        grid_spec=grid_spec,
        out_shape=jax.ShapeDtypeStruct((B, D, B), jnp.float32),
        compiler_params=pltpu.CompilerParams(
            dimension_semantics=("arbitrary",),
        ),
    )(idx, *([wT] * BI), e_bcast)
    return out_idj.transpose(0, 2, 1)


# speedup vs baseline: 1.0289x; 1.0289x over previous
import jax
import jax.numpy as jnp
from jax import lax
from jax.experimental import pallas as pl
from jax.experimental.pallas import tpu as pltpu

B = 1024
D = 32
BI = 128


def _body(idx_ref, *refs):
    col_refs = refs[:BI]
    eb_ref, out_ref = refs[BI], refs[BI + 1]
    i = pl.program_id(0)

    cols = []
    for t in range(BI):
        lane = lax.rem(idx_ref[i * BI + t], 128)
        onehot = lax.broadcasted_iota(jnp.int32, (D, 128), 1) == lane
        x = col_refs[t][...]
        cols.append(jnp.sum(jnp.where(onehot, x, 0.0), axis=1, keepdims=True))
    gT = jnp.concatenate(cols, axis=1)
    eye = (lax.broadcasted_iota(jnp.int32, (D, D), 0)
           == lax.broadcasted_iota(jnp.int32, (D, D), 1)).astype(jnp.float32)
    g = lax.dot_general(gT, eye, (((0,), (0,)), ((), ())),
                        preferred_element_type=jnp.float32)
    n = g * lax.rsqrt(jnp.sum(g * g, axis=1, keepdims=True))
    eTv = eb_ref[:, 0, :]
    for s in range(BI // 8):
        nsub = n[s * 8 : (s + 1) * 8, :]
        dsub = jnp.dot(nsub, eTv, preferred_element_type=jnp.float32)
        for d in range(D):
            out_ref[s * 8 : (s + 1) * 8, d, :] = (
                eb_ref[d] - nsub[:, d : d + 1] * dsub
            )


def kernel(batch_e_emb, batch_r_id, w_r):
    idx = batch_r_id.reshape(B).astype(jnp.int32)
    eT = batch_e_emb.T
    e_bcast = jnp.broadcast_to(eT[:, None, :], (D, 8, B))
    wT = w_r.T

    def col_spec(t):
        return pl.BlockSpec(
            (D, 128), lambda i, idx_ref, t=t: (0, idx_ref[i * BI + t] // 128)
        )

    grid_spec = pltpu.PrefetchScalarGridSpec(
        num_scalar_prefetch=1,
        grid=(B // BI,),
        in_specs=[col_spec(t) for t in range(BI)]
        + [pl.BlockSpec((D, 8, B), lambda i, idx_ref: (0, 0, 0))],
        out_specs=pl.BlockSpec((BI, D, B), lambda i, idx_ref: (i, 0, 0)),
        scratch_shapes=[],
    )
    out_idj = pl.pallas_call(
        _body,
        grid_spec=grid_spec,
        out_shape=jax.ShapeDtypeStruct((B, D, B), jnp.float32),
        compiler_params=pltpu.CompilerParams(
            dimension_semantics=("arbitrary",),
        ),
    )(idx, *([wT] * BI), e_bcast)
    return out_idj.transpose(0, 2, 1)
